# baseline (speedup 1.0000x reference)
import functools

import jax
import jax.numpy as jnp
from jax import lax
from jax.experimental import pallas as pl
from jax.experimental.pallas import tpu as pltpu
from jax.experimental.pallas import tpu_sc as plsc

_B, _K, _N, _D = 4096, 8, 1024, 256
_BB = 512


def _argmin_tc_body(x_ref, e_ref, idx_ref, gidx_ref, e2_ref):
    @pl.when(pl.program_id(0) == 0)
    def _():
        e2_ref[...] = 0.5 * jnp.sum(e_ref[...] * e_ref[...], axis=-1)

    lane_k = lax.broadcasted_iota(jnp.int32, (_BB, _K), 1)
    acc = jnp.zeros((_BB, _K), jnp.int32)
    for k in range(_K):
        xk = x_ref[:, k * _D:(k + 1) * _D]
        ek = e_ref[k]
        cross = lax.dot_general(xk, ek, (((1,), (1,)), ((), ())),
                                preferred_element_type=jnp.float32)
        score = e2_ref[k:k + 1, :] - cross
        idxk = jnp.argmin(score, axis=1, keepdims=True).astype(jnp.int32)
        acc = jnp.where(lane_k == k, idxk, acc)
    idx_ref[...] = acc
    gidx_ref[...] = acc + lane_k * _N


def _argmin_call(x, entries):
    return pl.pallas_call(
        _argmin_tc_body,
        grid=(_B // _BB,),
        in_specs=[
            pl.BlockSpec((_BB, _K * _D), lambda i: (i, 0)),
            pl.BlockSpec((_K, _N, _D), lambda i: (0, 0, 0)),
        ],
        out_specs=[
            pl.BlockSpec((_BB, _K), lambda i: (i, 0)),
            pl.BlockSpec((_BB, _K), lambda i: (i, 0)),
        ],
        out_shape=[
            jax.ShapeDtypeStruct((_B, _K), jnp.int32),
            jax.ShapeDtypeStruct((_B, _K), jnp.int32),
        ],
        scratch_shapes=[pltpu.VMEM((_K, _N), jnp.float32)],
    )(x, entries)


_ROWS = _B * _K
_CH = 128


def _gather_sc_body(nc, rpw, table_hbm, gidx_hbm, out_hbm,
                    idx_v, buf0, buf1, sem0, sem1):
    wid = lax.axis_index("s") * nc + lax.axis_index("c")
    base = wid * rpw
    pltpu.sync_copy(gidx_hbm.at[pl.ds(base, rpw)], idx_v)
    bufs = (buf0, buf1)
    sems = (sem0, sem1)
    nch = rpw // _CH
    cps = [None, None]
    cps[0] = pltpu.async_copy(
        table_hbm.at[idx_v.at[pl.ds(0, _CH)]], bufs[0], sems[0])
    for c in range(nch):
        cur = c % 2
        nxt = (c + 1) % 2
        if c + 1 < nch:
            cps[nxt] = pltpu.async_copy(
                table_hbm.at[idx_v.at[pl.ds((c + 1) * _CH, _CH)]],
                bufs[nxt], sems[nxt])
        cps[cur].wait()
        pltpu.sync_copy(bufs[cur], out_hbm.at[pl.ds(base + c * _CH, _CH)])


def _gather_call(table, gidx_flat):
    info = plsc.get_sparse_core_info()
    nw = info.num_cores * info.num_subcores
    rpw = _ROWS // nw
    fn = pl.kernel(
        functools.partial(_gather_sc_body, info.num_cores, rpw),
        out_type=jax.ShapeDtypeStruct((_ROWS, _D), jnp.float32),
        mesh=plsc.VectorSubcoreMesh(core_axis_name="c", subcore_axis_name="s"),
        scratch_types=[
            pltpu.VMEM((rpw,), jnp.int32),
            pltpu.VMEM((_CH, _D), jnp.float32),
            pltpu.VMEM((_CH, _D), jnp.float32),
            pltpu.SemaphoreType.DMA,
            pltpu.SemaphoreType.DMA,
        ],
    )
    return fn(table, gidx_flat)


def kernel(x, entries):
    idx, gidx = _argmin_call(x.reshape(_B, _K * _D), entries)
    table = entries.reshape(_K * _N, _D)
    q = _gather_call(table, gidx.reshape(_ROWS))
    return q.reshape(_B, _K, _D), idx


# speedup vs baseline: 1.1606x; 1.1606x over previous
import functools

import jax
import jax.numpy as jnp
from jax import lax
from jax.experimental import pallas as pl
from jax.experimental.pallas import tpu as pltpu
from jax.experimental.pallas import tpu_sc as plsc

_B, _K, _N, _D = 4096, 8, 1024, 256
_BB = 512


def _argmin_tc_body(x_ref, e_ref, idx_ref, gidx_ref, e2_ref):
    @pl.when(pl.program_id(0) == 0)
    def _():
        e2_ref[...] = 0.5 * jnp.sum(e_ref[...] * e_ref[...], axis=-1)

    lane_k = lax.broadcasted_iota(jnp.int32, (_BB, _K), 1)
    acc = jnp.zeros((_BB, _K), jnp.int32)
    for k in range(_K):
        xk = x_ref[:, k, :]
        ek = e_ref[k]
        cross = lax.dot_general(xk, ek, (((1,), (1,)), ((), ())),
                                preferred_element_type=jnp.float32)
        score = e2_ref[k:k + 1, :] - cross
        idxk = jnp.argmin(score, axis=1, keepdims=True).astype(jnp.int32)
        acc = jnp.where(lane_k == k, idxk, acc)
    idx_ref[...] = acc
    gidx_ref[...] = acc + lane_k * _N


def _argmin_call(x, entries):
    return pl.pallas_call(
        _argmin_tc_body,
        grid=(_B // _BB,),
        in_specs=[
            pl.BlockSpec((_BB, _K, _D), lambda i: (i, 0, 0)),
            pl.BlockSpec((_K, _N, _D), lambda i: (0, 0, 0)),
        ],
        out_specs=[
            pl.BlockSpec((_BB, _K), lambda i: (i, 0)),
            pl.BlockSpec((_BB, _K), lambda i: (i, 0)),
        ],
        out_shape=[
            jax.ShapeDtypeStruct((_B, _K), jnp.int32),
            jax.ShapeDtypeStruct((_B, _K), jnp.int32),
        ],
        scratch_shapes=[pltpu.VMEM((_K, _N), jnp.float32)],
    )(x, entries)


_ROWS = _B * _K
_CH = 128


def _gather_sc_body(nc, rpw, table_hbm, gidx_hbm, out_hbm,
                    idx_v, buf0, buf1, sem0, sem1):
    wid = lax.axis_index("s") * nc + lax.axis_index("c")
    base = wid * rpw
    pltpu.sync_copy(gidx_hbm.at[pl.ds(base, rpw)], idx_v)
    bufs = (buf0, buf1)
    sems = (sem0, sem1)
    nch = rpw // _CH
    cps = [None, None]
    cps[0] = pltpu.async_copy(
        table_hbm.at[idx_v.at[pl.ds(0, _CH)]], bufs[0], sems[0])
    for c in range(nch):
        cur = c % 2
        nxt = (c + 1) % 2
        if c + 1 < nch:
            cps[nxt] = pltpu.async_copy(
                table_hbm.at[idx_v.at[pl.ds((c + 1) * _CH, _CH)]],
                bufs[nxt], sems[nxt])
        cps[cur].wait()
        pltpu.sync_copy(bufs[cur], out_hbm.at[pl.ds(base + c * _CH, _CH)])


def _gather_call(table, gidx_flat):
    info = plsc.get_sparse_core_info()
    nw = info.num_cores * info.num_subcores
    rpw = _ROWS // nw
    fn = pl.kernel(
        functools.partial(_gather_sc_body, info.num_cores, rpw),
        out_type=jax.ShapeDtypeStruct((_ROWS, _D), jnp.float32),
        mesh=plsc.VectorSubcoreMesh(core_axis_name="c", subcore_axis_name="s"),
        scratch_types=[
            pltpu.VMEM((rpw,), jnp.int32),
            pltpu.VMEM((_CH, _D), jnp.float32),
            pltpu.VMEM((_CH, _D), jnp.float32),
            pltpu.SemaphoreType.DMA,
            pltpu.SemaphoreType.DMA,
        ],
    )
    return fn(table, gidx_flat)


def kernel(x, entries):
    idx, gidx = _argmin_call(x, entries)
    table = entries.reshape(_K * _N, _D)
    q = _gather_call(table, gidx.reshape(_ROWS))
    return q.reshape(_B, _K, _D), idx


# speedup vs baseline: 1.1664x; 1.0051x over previous
import functools

import jax
import jax.numpy as jnp
from jax import lax
from jax.experimental import pallas as pl
from jax.experimental.pallas import tpu as pltpu
from jax.experimental.pallas import tpu_sc as plsc

_B, _K, _N, _D = 4096, 8, 1024, 256
_BB = 512


def _argmin_tc_body(x_ref, e_ref, idx_ref, gidx_ref, e2_ref):
    @pl.when(pl.program_id(0) == 0)
    def _():
        e2_ref[...] = 0.5 * jnp.sum(e_ref[...] * e_ref[...], axis=-1)

    lane_k = lax.broadcasted_iota(jnp.int32, (_BB, _K), 1)
    acc = jnp.zeros((_BB, _K), jnp.int32)
    for k in range(_K):
        xk = x_ref[:, k, :]
        ek = e_ref[k]
        cross = lax.dot_general(xk, ek, (((1,), (1,)), ((), ())),
                                preferred_element_type=jnp.float32)
        score = e2_ref[k:k + 1, :] - cross
        idxk = jnp.argmin(score, axis=1, keepdims=True).astype(jnp.int32)
        acc = jnp.where(lane_k == k, idxk, acc)
    idx_ref[...] = acc
    gidx_ref[...] = acc + lane_k * _N


def _argmin_call(x, entries):
    return pl.pallas_call(
        _argmin_tc_body,
        grid=(_B // _BB,),
        in_specs=[
            pl.BlockSpec((_BB, _K, _D), lambda i: (i, 0, 0)),
            pl.BlockSpec((_K, _N, _D), lambda i: (0, 0, 0)),
        ],
        out_specs=[
            pl.BlockSpec((_BB, _K), lambda i: (i, 0)),
            pl.BlockSpec((_BB, _K), lambda i: (i, 0)),
        ],
        out_shape=[
            jax.ShapeDtypeStruct((_B, _K), jnp.int32),
            jax.ShapeDtypeStruct((_B, _K), jnp.int32),
        ],
        scratch_shapes=[pltpu.VMEM((_K, _N), jnp.float32)],
    )(x, entries)


_ROWS = _B * _K
_CH = 128


_NB = 3


def _gather_sc_body(nc, rpw, table_hbm, gidx_hbm, out_hbm,
                    idx_v, buf0, buf1, buf2, gs0, gs1, gs2, ss0, ss1, ss2):
    wid = lax.axis_index("s") * nc + lax.axis_index("c")
    base = wid * rpw
    pltpu.sync_copy(gidx_hbm.at[pl.ds(base, rpw)], idx_v)
    bufs = (buf0, buf1, buf2)
    gs = (gs0, gs1, gs2)
    ss = (ss0, ss1, ss2)
    nch = rpw // _CH

    def gather(c):
        return pltpu.async_copy(
            table_hbm.at[idx_v.at[pl.ds(c * _CH, _CH)]],
            bufs[c % _NB], gs[c % _NB])

    g = [None] * nch
    s = [None] * nch
    for c in range(_NB - 1):
        g[c] = gather(c)
    for c in range(nch):
        f = c + _NB - 1
        if f < nch:
            if c >= 1:
                s[c - 1].wait()
            g[f] = gather(f)
        g[c].wait()
        s[c] = pltpu.async_copy(
            bufs[c % _NB], out_hbm.at[pl.ds(base + c * _CH, _CH)], ss[c % _NB])
    for c in range(max(0, nch - _NB), nch):
        s[c].wait()


def _gather_call(table, gidx_flat):
    info = plsc.get_sparse_core_info()
    nw = info.num_cores * info.num_subcores
    rpw = _ROWS // nw
    fn = pl.kernel(
        functools.partial(_gather_sc_body, info.num_cores, rpw),
        out_type=jax.ShapeDtypeStruct((_ROWS, _D), jnp.float32),
        mesh=plsc.VectorSubcoreMesh(core_axis_name="c", subcore_axis_name="s"),
        scratch_types=[
            pltpu.VMEM((rpw,), jnp.int32),
            pltpu.VMEM((_CH, _D), jnp.float32),
            pltpu.VMEM((_CH, _D), jnp.float32),
            pltpu.VMEM((_CH, _D), jnp.float32),
            pltpu.SemaphoreType.DMA,
            pltpu.SemaphoreType.DMA,
            pltpu.SemaphoreType.DMA,
            pltpu.SemaphoreType.DMA,
            pltpu.SemaphoreType.DMA,
            pltpu.SemaphoreType.DMA,
        ],
    )
    return fn(table, gidx_flat)


def kernel(x, entries):
    idx, gidx = _argmin_call(x, entries)
    table = entries.reshape(_K * _N, _D)
    q = _gather_call(table, gidx.reshape(_ROWS))
    return q.reshape(_B, _K, _D), idx
